# baseline (speedup 1.0000x reference)
import functools

import jax
import jax.numpy as jnp
from jax import lax
from jax.experimental import pallas as pl
from jax.experimental.pallas import tpu as pltpu
from jax.experimental.pallas import tpu_sc as plsc

N = 10000
E = 320000
F = 128
R = 5

ROW_BLK = 1000

NC = 2
NS = 16
L = 16
NW = NC * NS
SB = 128
NSB = E // SB
RPT = 624


def _rel_mm_body(x_ref, w_ref, o_ref):
    o_ref[0] = jnp.dot(x_ref[...], w_ref[0],
                       preferred_element_type=jnp.float32)


def rel_matmul(x, w_all):
    J, Din, Fo = w_all.shape
    n = x.shape[0]
    grid = (J, n // ROW_BLK)
    return pl.pallas_call(
        _rel_mm_body,
        grid=grid,
        in_specs=[
            pl.BlockSpec((ROW_BLK, Din), lambda j, i: (i, 0)),
            pl.BlockSpec((1, Din, Fo), lambda j, i: (j, 0, 0)),
        ],
        out_specs=pl.BlockSpec((1, ROW_BLK, Fo), lambda j, i: (j, i, 0)),
        out_shape=jax.ShapeDtypeStruct((J, n, Fo), jnp.float32),
    )(x, w_all)


def _epilogue_body(agg_ref, root_ref, s_ref, c_ref, a_ref, res_ref, o_ref):
    agg = agg_ref[0] + agg_ref[1]
    t = (agg + root_ref[...]) * s_ref[...] + c_ref[...]
    t = jnp.where(t >= 0, t, a_ref[0, 0] * t)
    o_ref[...] = t + res_ref[...]


def _epilogue_body_nores(agg_ref, root_ref, s_ref, c_ref, a_ref, o_ref):
    agg = agg_ref[0] + agg_ref[1]
    t = (agg + root_ref[...]) * s_ref[...] + c_ref[...]
    o_ref[...] = jnp.where(t >= 0, t, a_ref[0, 0] * t)


def conv_epilogue(agg2, root, s, c, a, res=None):
    n = root.shape[0]
    grid = (n // ROW_BLK,)
    agg_spec = pl.BlockSpec((NC, ROW_BLK, F), lambda i: (0, i, 0))
    row_spec = pl.BlockSpec((ROW_BLK, F), lambda i: (i, 0))
    vec_spec = pl.BlockSpec((1, F), lambda i: (0, 0))
    scal_spec = pl.BlockSpec((1, 1), lambda i: (0, 0))
    if res is None:
        return pl.pallas_call(
            _epilogue_body_nores,
            grid=grid,
            in_specs=[agg_spec, row_spec, vec_spec, vec_spec, scal_spec],
            out_specs=row_spec,
            out_shape=jax.ShapeDtypeStruct((n, F), jnp.float32),
        )(agg2, root, s, c, a)
    return pl.pallas_call(
        _epilogue_body,
        grid=grid,
        in_specs=[agg_spec, row_spec, vec_spec, vec_spec, scal_spec, row_spec],
        out_specs=row_spec,
        out_shape=jax.ShapeDtypeStruct((n, F), jnp.float32),
    )(agg2, root, s, c, a, res)


def _mm_bias_body(x_ref, w_ref, b_ref, o_ref):
    o_ref[...] = jnp.dot(x_ref[...], w_ref[...],
                         preferred_element_type=jnp.float32) + b_ref[...]


def mm_bias(x, w, b):
    n, Din = x.shape
    Fo = w.shape[1]
    return pl.pallas_call(
        _mm_bias_body,
        grid=(n // ROW_BLK,),
        in_specs=[
            pl.BlockSpec((ROW_BLK, Din), lambda i: (i, 0)),
            pl.BlockSpec((Din, Fo), lambda i: (0, 0)),
            pl.BlockSpec((1, Fo), lambda i: (0, 0)),
        ],
        out_specs=pl.BlockSpec((ROW_BLK, Fo), lambda i: (i, 0)),
        out_shape=jax.ShapeDtypeStruct((n, Fo), jnp.float32),
    )(x, w, b)


SBT = 80
NSBP = NW * SBT
NB2 = SBT // 2


def _scale_block(rows_v, coef_v):
    def scale_body(g, c2):
        rb = g * L
        cvec = coef_v[pl.ds(rb, L)]
        for r in range(L):
            bvec = cvec.at[jnp.full((L,), r, jnp.int32)].get(
                mode="promise_in_bounds")
            for q in range(F // L):
                rows_v[rb + r, pl.ds(q * L, L)] = (
                    rows_v[rb + r, pl.ds(q * L, L)] * bvec)
        return c2

    lax.fori_loop(0, SB // L, scale_body, 0)


def _sc_agg_body(hs, gidx, dst, coef, out,
                 gidx0, dst0, coef0, gidx1, dst1, coef1,
                 rows0, rows1, agg_sh,
                 gsem0, gsem1, ssem0, ssem1):
    cid = lax.axis_index("c")
    sid = lax.axis_index("s")
    w = sid * NC + cid

    zero = jnp.zeros((L,), jnp.float32)

    def zero_body(r, carry):
        for g in range(F // L):
            rows0[r, pl.ds(g * L, L)] = zero
        return carry

    lax.fori_loop(0, SB, zero_body, 0)

    base = sid * RPT
    for k in range(4):
        pltpu.sync_copy(rows0.at[pl.ds(0, SB)],
                        agg_sh.at[pl.ds(base + k * SB, SB)])
    pltpu.sync_copy(rows0.at[pl.ds(0, 112)],
                    agg_sh.at[pl.ds(base + 4 * SB, 112)])

    @pl.when(sid == NS - 1)
    def _zero_tail():
        pltpu.sync_copy(rows0.at[pl.ds(0, 16)],
                        agg_sh.at[pl.ds(NS * RPT, 16)])

    ebs = w * SBT * SB

    def load_idx(j, gv, dv, cv):
        eb = ebs + j * SB
        pltpu.sync_copy(gidx.at[pl.ds(eb, SB)], gv)
        pltpu.sync_copy(dst.at[pl.ds(eb, SB)], dv)
        pltpu.sync_copy(coef.at[pl.ds(eb, SB)], cv)

    def gather(gv, rows, sem):
        return pltpu.async_copy(hs.at[gv], rows, sem)

    def scatter(dv, rows, sem):
        return pltpu.async_copy(rows, agg_sh.at[dv], sem, add=True)

    load_idx(0, gidx0, dst0, coef0)
    load_idx(1, gidx1, dst1, coef1)
    gather(gidx0, rows0, gsem0)
    gather(gidx1, rows1, gsem1)

    def body2(i, carry):
        a = 2 * i
        b = a + 1
        pltpu.make_async_copy(hs.at[gidx0], rows0, gsem0).wait()
        _scale_block(rows0, coef0)
        scatter(dst0, rows0, ssem0)
        pltpu.make_async_copy(hs.at[gidx1], rows1, gsem1).wait()
        _scale_block(rows1, coef1)
        scatter(dst1, rows1, ssem1)
        pltpu.make_async_copy(rows0, agg_sh.at[dst0], ssem0).wait()

        @pl.when(a + 2 < SBT)
        def _g0():
            load_idx(a + 2, gidx0, dst0, coef0)
            gather(gidx0, rows0, gsem0)

        pltpu.make_async_copy(rows1, agg_sh.at[dst1], ssem1).wait()

        @pl.when(b + 2 < SBT)
        def _g1():
            load_idx(b + 2, gidx1, dst1, coef1)
            gather(gidx1, rows1, gsem1)

        return carry

    lax.fori_loop(0, NB2, body2, 0)
    plsc.subcore_barrier()

    for k in range(4):
        pltpu.sync_copy(agg_sh.at[pl.ds(base + k * SB, SB)],
                        out.at[cid, pl.ds(base + k * SB, SB), :])
    pltpu.sync_copy(agg_sh.at[pl.ds(base + 4 * SB, 112)],
                    out.at[cid, pl.ds(base + 4 * SB, 112), :])

    @pl.when(sid == NS - 1)
    def _out_tail():
        pltpu.sync_copy(agg_sh.at[pl.ds(NS * RPT, 16)],
                        out.at[cid, pl.ds(NS * RPT, 16), :])


def sc_aggregate(hs_flat, gidx2, dst2, coef2):
    mesh = plsc.VectorSubcoreMesh(core_axis_name="c", subcore_axis_name="s",
                                  num_cores=NC, num_subcores=NS)
    return pl.kernel(
        _sc_agg_body,
        out_type=jax.ShapeDtypeStruct((NC, N, F), jnp.float32),
        mesh=mesh,
        scratch_types=[
            pltpu.VMEM((SB,), jnp.int32),
            pltpu.VMEM((SB,), jnp.int32),
            pltpu.VMEM((SB,), jnp.float32),
            pltpu.VMEM((SB,), jnp.int32),
            pltpu.VMEM((SB,), jnp.int32),
            pltpu.VMEM((SB,), jnp.float32),
            pltpu.VMEM((SB, F), jnp.float32),
            pltpu.VMEM((SB, F), jnp.float32),
            pltpu.VMEM_SHARED((N, F), jnp.float32),
            pltpu.SemaphoreType.DMA,
            pltpu.SemaphoreType.DMA,
            pltpu.SemaphoreType.DMA,
            pltpu.SemaphoreType.DMA,
        ],
    )(hs_flat, gidx2, dst2, coef2)


def _bn_consts(p):
    s = p["bn_g"] / jnp.sqrt(p["bn_rv"] + 1e-5)
    c = p["bn_b"] + (p["b"] - p["bn_rm"]) * s
    return s.reshape(1, F), c.reshape(1, F), p["prelu"].reshape(1, 1)


def _conv_block(h, gidx, dst, coef, p, residual):
    w_all = jnp.concatenate([p["w_rel"], p["w_root"][None]], axis=0)
    hs6 = rel_matmul(h, w_all)
    agg2 = sc_aggregate(hs6.reshape((R + 1) * N, F), gidx, dst, coef)
    s, c, a = _bn_consts(p)
    return conv_epilogue(agg2, hs6[R], s, c, a, h if residual else None)


def kernel(x, edge_index, edge_attr, params):
    src = edge_index[0].astype(jnp.int32)
    dst = edge_index[1].astype(jnp.int32)
    et = edge_attr[:, 4].astype(jnp.int32)

    cnt = jnp.zeros((N * R,), jnp.float32).at[dst * R + et].add(1.0)
    coef = 1.0 / jnp.maximum(cnt[dst * R + et], 1.0)

    pad = NSBP * SB - E
    gidx = jnp.concatenate([et * N + src, jnp.zeros((pad,), jnp.int32)])
    coef = jnp.concatenate([coef, jnp.zeros((pad,), jnp.float32)])
    dst = jnp.concatenate([dst, jnp.zeros((pad,), jnp.int32)])

    h = _conv_block(x, gidx, dst, coef, params["enc0"], residual=False)
    h = _conv_block(h, gidx, dst, coef, params["enc1"], residual=True)

    wm = jnp.concatenate([params["fc_mu"]["w"], params["fc_logvar"]["w"]], axis=1)
    bm = jnp.concatenate([params["fc_mu"]["b"], params["fc_logvar"]["b"]])
    ml = mm_bias(h, wm, bm.reshape(1, -1))
    mu, logvar = ml[:, :64], ml[:, 64:]

    d = jnp.concatenate([mu, x], axis=1)
    d = _conv_block(d, gidx, dst, coef, params["dec0"], residual=False)
    d = _conv_block(d, gidx, dst, coef, params["dec1"], residual=True)
    out = mm_bias(d, params["fc_out"]["w"], params["fc_out"]["b"].reshape(1, -1))
    return (out, mu, logvar)


# speedup vs baseline: 1.5871x; 1.5871x over previous
import functools

import jax
import jax.numpy as jnp
from jax import lax
from jax.experimental import pallas as pl
from jax.experimental.pallas import tpu as pltpu
from jax.experimental.pallas import tpu_sc as plsc

N = 10000
E = 320000
F = 128
R = 5

ROW_BLK = 1000

NC = 2
NS = 16
L = 16
NW = NC * NS
SB = 128
NSB = E // SB
RPT = 624


def _rel_mm_body(x_ref, w_ref, o_ref):
    o_ref[0] = jnp.dot(x_ref[...], w_ref[0],
                       preferred_element_type=jnp.float32)


def rel_matmul(x, w_all):
    J, Din, Fo = w_all.shape
    n = x.shape[0]
    grid = (J, n // ROW_BLK)
    return pl.pallas_call(
        _rel_mm_body,
        grid=grid,
        in_specs=[
            pl.BlockSpec((ROW_BLK, Din), lambda j, i: (i, 0)),
            pl.BlockSpec((1, Din, Fo), lambda j, i: (j, 0, 0)),
        ],
        out_specs=pl.BlockSpec((1, ROW_BLK, Fo), lambda j, i: (j, i, 0)),
        out_shape=jax.ShapeDtypeStruct((J, n, Fo), jnp.float32),
    )(x, w_all)


def _epilogue_body(agg_ref, root_ref, s_ref, c_ref, a_ref, res_ref, o_ref):
    agg = agg_ref[0] + agg_ref[1]
    t = (agg + root_ref[...]) * s_ref[...] + c_ref[...]
    t = jnp.where(t >= 0, t, a_ref[0, 0] * t)
    o_ref[...] = t + res_ref[...]


def _epilogue_body_nores(agg_ref, root_ref, s_ref, c_ref, a_ref, o_ref):
    agg = agg_ref[0] + agg_ref[1]
    t = (agg + root_ref[...]) * s_ref[...] + c_ref[...]
    o_ref[...] = jnp.where(t >= 0, t, a_ref[0, 0] * t)


def conv_epilogue(agg2, root, s, c, a, res=None):
    n = root.shape[0]
    grid = (n // ROW_BLK,)
    agg_spec = pl.BlockSpec((NC, ROW_BLK, F), lambda i: (0, i, 0))
    row_spec = pl.BlockSpec((ROW_BLK, F), lambda i: (i, 0))
    vec_spec = pl.BlockSpec((1, F), lambda i: (0, 0))
    scal_spec = pl.BlockSpec((1, 1), lambda i: (0, 0))
    if res is None:
        return pl.pallas_call(
            _epilogue_body_nores,
            grid=grid,
            in_specs=[agg_spec, row_spec, vec_spec, vec_spec, scal_spec],
            out_specs=row_spec,
            out_shape=jax.ShapeDtypeStruct((n, F), jnp.float32),
        )(agg2, root, s, c, a)
    return pl.pallas_call(
        _epilogue_body,
        grid=grid,
        in_specs=[agg_spec, row_spec, vec_spec, vec_spec, scal_spec, row_spec],
        out_specs=row_spec,
        out_shape=jax.ShapeDtypeStruct((n, F), jnp.float32),
    )(agg2, root, s, c, a, res)


def _mm_bias_body(x_ref, w_ref, b_ref, o_ref):
    o_ref[...] = jnp.dot(x_ref[...], w_ref[...],
                         preferred_element_type=jnp.float32) + b_ref[...]


def mm_bias(x, w, b):
    n, Din = x.shape
    Fo = w.shape[1]
    return pl.pallas_call(
        _mm_bias_body,
        grid=(n // ROW_BLK,),
        in_specs=[
            pl.BlockSpec((ROW_BLK, Din), lambda i: (i, 0)),
            pl.BlockSpec((Din, Fo), lambda i: (0, 0)),
            pl.BlockSpec((1, Fo), lambda i: (0, 0)),
        ],
        out_specs=pl.BlockSpec((ROW_BLK, Fo), lambda i: (i, 0)),
        out_shape=jax.ShapeDtypeStruct((n, Fo), jnp.float32),
    )(x, w, b)


SBT = 80
NSBP = NW * SBT
NB2 = SBT // 2


def _scale_block(rows_v, coef_v):
    def scale_body(g, c2):
        rb = g * L
        cvec = coef_v[pl.ds(rb, L)]
        for r in range(L):
            bvec = cvec.at[jnp.full((L,), r, jnp.int32)].get(
                mode="promise_in_bounds")
            for q in range(F // L):
                rows_v[rb + r, pl.ds(q * L, L)] = (
                    rows_v[rb + r, pl.ds(q * L, L)] * bvec)
        return c2

    lax.fori_loop(0, SB // L, scale_body, 0)


HB = 512
EPT = NSBP * SB // NW
HPS = HB * SB // NS
HCB = EPT // SB


def _sc_hist_body(hidx, out0, out1, hbuf0, hbuf1, ones_v, zeros_v, acc_sh,
                  ssem0, ssem1):
    cid = lax.axis_index("c")
    sid = lax.axis_index("s")
    w = sid * NC + cid

    one = jnp.ones((L,), jnp.float32)
    zero = jnp.zeros((L,), jnp.float32)

    def fill_body(g, carry):
        ones_v[pl.ds(g * L, L)] = one
        zeros_v[pl.ds(g * L, L)] = zero
        return carry

    lax.fori_loop(0, SB // L, fill_body, 0)

    base = sid * HPS
    for k in range(HPS // SB):
        pltpu.sync_copy(zeros_v, acc_sh.at[pl.ds(base + k * SB, SB)])
    plsc.subcore_barrier()

    ebs = w * EPT

    def load_idx(j, hb):
        pltpu.sync_copy(hidx.at[pl.ds(ebs + j * SB, SB)], hb)

    def scat(hb, sem):
        return pltpu.async_copy(ones_v, acc_sh.at[hb], sem, add=True)

    load_idx(0, hbuf0)
    scat(hbuf0, ssem0)
    load_idx(1, hbuf1)
    scat(hbuf1, ssem1)

    def body2(i, carry):
        a = 2 * i
        b = a + 1
        pltpu.make_async_copy(ones_v, acc_sh.at[hbuf0], ssem0).wait()

        @pl.when(a + 2 < HCB)
        def _s0():
            load_idx(a + 2, hbuf0)
            scat(hbuf0, ssem0)

        pltpu.make_async_copy(ones_v, acc_sh.at[hbuf1], ssem1).wait()

        @pl.when(b + 2 < HCB)
        def _s1():
            load_idx(b + 2, hbuf1)
            scat(hbuf1, ssem1)

        return carry

    lax.fori_loop(0, HCB // 2, body2, 0)
    plsc.subcore_barrier()

    @pl.when(cid == 0)
    def _co0():
        pltpu.sync_copy(acc_sh.at[pl.ds(base, HPS)],
                        out0.at[pl.ds(base, HPS)])

    @pl.when(cid == 1)
    def _co1():
        pltpu.sync_copy(acc_sh.at[pl.ds(base, HPS)],
                        out1.at[pl.ds(base, HPS)])


def sc_histogram(hidx):
    mesh = plsc.VectorSubcoreMesh(core_axis_name="c", subcore_axis_name="s",
                                  num_cores=NC, num_subcores=NS)
    return pl.kernel(
        _sc_hist_body,
        out_type=(jax.ShapeDtypeStruct((HB * SB,), jnp.float32),
                  jax.ShapeDtypeStruct((HB * SB,), jnp.float32)),
        mesh=mesh,
        scratch_types=[
            pltpu.VMEM((SB,), jnp.int32),
            pltpu.VMEM((SB,), jnp.int32),
            pltpu.VMEM((SB,), jnp.float32),
            pltpu.VMEM((SB,), jnp.float32),
            pltpu.VMEM_SHARED((HB * SB,), jnp.float32),
            pltpu.SemaphoreType.DMA,
            pltpu.SemaphoreType.DMA,
        ],
    )(hidx)


def _merge_body(p0_ref, p1_ref, o_ref):
    s = p0_ref[...] + p1_ref[...]
    row = lax.broadcasted_iota(jnp.int32, (HB, SB), 0)
    col = lax.broadcasted_iota(jnp.int32, (HB, SB), 1)
    b = row * SB + col
    o_ref[...] = jnp.where(b < N * R, 1.0 / jnp.maximum(s, 1.0), 0.0)


def tc_merge_recip(p0, p1):
    return pl.pallas_call(
        _merge_body,
        out_shape=jax.ShapeDtypeStruct((HB, SB), jnp.float32),
    )(p0, p1)


def _sc_coef_body(recip, hidx, out, hbuf0, hbuf1, cbuf0, cbuf1,
                  gsem0, gsem1):
    cid = lax.axis_index("c")
    sid = lax.axis_index("s")
    w = sid * NC + cid
    ebs = w * EPT

    def load_idx(j, hb):
        pltpu.sync_copy(hidx.at[pl.ds(ebs + j * SB, SB)], hb)

    def gather(hb, cb, sem):
        return pltpu.async_copy(recip.at[hb], cb, sem)

    def flush(j, cb):
        pltpu.sync_copy(cb, out.at[pl.ds(ebs + j * SB, SB)])

    load_idx(0, hbuf0)
    gather(hbuf0, cbuf0, gsem0)
    load_idx(1, hbuf1)
    gather(hbuf1, cbuf1, gsem1)

    def body2(i, carry):
        a = 2 * i
        b = a + 1
        pltpu.make_async_copy(recip.at[hbuf0], cbuf0, gsem0).wait()
        flush(a, cbuf0)

        @pl.when(a + 2 < SBT)
        def _g0():
            load_idx(a + 2, hbuf0)
            gather(hbuf0, cbuf0, gsem0)

        pltpu.make_async_copy(recip.at[hbuf1], cbuf1, gsem1).wait()
        flush(b, cbuf1)

        @pl.when(b + 2 < SBT)
        def _g1():
            load_idx(b + 2, hbuf1)
            gather(hbuf1, cbuf1, gsem1)

        return carry

    lax.fori_loop(0, NB2, body2, 0)


def sc_coef(recip_flat, hidx):
    mesh = plsc.VectorSubcoreMesh(core_axis_name="c", subcore_axis_name="s",
                                  num_cores=NC, num_subcores=NS)
    return pl.kernel(
        _sc_coef_body,
        out_type=jax.ShapeDtypeStruct((NSBP * SB,), jnp.float32),
        mesh=mesh,
        scratch_types=[
            pltpu.VMEM((SB,), jnp.int32),
            pltpu.VMEM((SB,), jnp.int32),
            pltpu.VMEM((SB,), jnp.float32),
            pltpu.VMEM((SB,), jnp.float32),
            pltpu.SemaphoreType.DMA,
            pltpu.SemaphoreType.DMA,
        ],
    )(recip_flat, hidx)


def _sc_agg_body(hs, gidx, dst, coef, out,
                 gidx0, dst0, coef0, gidx1, dst1, coef1,
                 rows0, rows1, agg_sh,
                 gsem0, gsem1, ssem0, ssem1):
    cid = lax.axis_index("c")
    sid = lax.axis_index("s")
    w = sid * NC + cid

    zero = jnp.zeros((L,), jnp.float32)

    def zero_body(r, carry):
        for g in range(F // L):
            rows0[r, pl.ds(g * L, L)] = zero
        return carry

    lax.fori_loop(0, SB, zero_body, 0)

    base = sid * RPT
    for k in range(4):
        pltpu.sync_copy(rows0.at[pl.ds(0, SB)],
                        agg_sh.at[pl.ds(base + k * SB, SB)])
    pltpu.sync_copy(rows0.at[pl.ds(0, 112)],
                    agg_sh.at[pl.ds(base + 4 * SB, 112)])

    @pl.when(sid == NS - 1)
    def _zero_tail():
        pltpu.sync_copy(rows0.at[pl.ds(0, 16)],
                        agg_sh.at[pl.ds(NS * RPT, 16)])

    plsc.subcore_barrier()

    ebs = w * SBT * SB

    def load_idx(j, gv, dv, cv):
        eb = ebs + j * SB
        pltpu.sync_copy(gidx.at[pl.ds(eb, SB)], gv)
        pltpu.sync_copy(dst.at[pl.ds(eb, SB)], dv)
        pltpu.sync_copy(coef.at[pl.ds(eb, SB)], cv)

    def gather(gv, rows, sem):
        return pltpu.async_copy(hs.at[gv], rows, sem)

    def scatter(dv, rows, sem):
        return pltpu.async_copy(rows, agg_sh.at[dv], sem, add=True)

    load_idx(0, gidx0, dst0, coef0)
    load_idx(1, gidx1, dst1, coef1)
    gather(gidx0, rows0, gsem0)
    gather(gidx1, rows1, gsem1)

    def body2(i, carry):
        a = 2 * i
        b = a + 1
        pltpu.make_async_copy(hs.at[gidx0], rows0, gsem0).wait()
        _scale_block(rows0, coef0)
        scatter(dst0, rows0, ssem0)
        pltpu.make_async_copy(hs.at[gidx1], rows1, gsem1).wait()
        _scale_block(rows1, coef1)
        scatter(dst1, rows1, ssem1)
        pltpu.make_async_copy(rows0, agg_sh.at[dst0], ssem0).wait()

        @pl.when(a + 2 < SBT)
        def _g0():
            load_idx(a + 2, gidx0, dst0, coef0)
            gather(gidx0, rows0, gsem0)

        pltpu.make_async_copy(rows1, agg_sh.at[dst1], ssem1).wait()

        @pl.when(b + 2 < SBT)
        def _g1():
            load_idx(b + 2, gidx1, dst1, coef1)
            gather(gidx1, rows1, gsem1)

        return carry

    lax.fori_loop(0, NB2, body2, 0)
    plsc.subcore_barrier()

    for k in range(4):
        pltpu.sync_copy(agg_sh.at[pl.ds(base + k * SB, SB)],
                        out.at[cid, pl.ds(base + k * SB, SB), :])
    pltpu.sync_copy(agg_sh.at[pl.ds(base + 4 * SB, 112)],
                    out.at[cid, pl.ds(base + 4 * SB, 112), :])

    @pl.when(sid == NS - 1)
    def _out_tail():
        pltpu.sync_copy(agg_sh.at[pl.ds(NS * RPT, 16)],
                        out.at[cid, pl.ds(NS * RPT, 16), :])


def sc_aggregate(hs_flat, gidx2, dst2, coef2):
    mesh = plsc.VectorSubcoreMesh(core_axis_name="c", subcore_axis_name="s",
                                  num_cores=NC, num_subcores=NS)
    return pl.kernel(
        _sc_agg_body,
        out_type=jax.ShapeDtypeStruct((NC, N, F), jnp.float32),
        mesh=mesh,
        scratch_types=[
            pltpu.VMEM((SB,), jnp.int32),
            pltpu.VMEM((SB,), jnp.int32),
            pltpu.VMEM((SB,), jnp.float32),
            pltpu.VMEM((SB,), jnp.int32),
            pltpu.VMEM((SB,), jnp.int32),
            pltpu.VMEM((SB,), jnp.float32),
            pltpu.VMEM((SB, F), jnp.float32),
            pltpu.VMEM((SB, F), jnp.float32),
            pltpu.VMEM_SHARED((N, F), jnp.float32),
            pltpu.SemaphoreType.DMA,
            pltpu.SemaphoreType.DMA,
            pltpu.SemaphoreType.DMA,
            pltpu.SemaphoreType.DMA,
        ],
    )(hs_flat, gidx2, dst2, coef2)


def _bn_consts(p):
    s = p["bn_g"] / jnp.sqrt(p["bn_rv"] + 1e-5)
    c = p["bn_b"] + (p["b"] - p["bn_rm"]) * s
    return s.reshape(1, F), c.reshape(1, F), p["prelu"].reshape(1, 1)


def _conv_block(h, gidx, dst, coef, p, residual):
    w_all = jnp.concatenate([p["w_rel"], p["w_root"][None]], axis=0)
    hs6 = rel_matmul(h, w_all)
    agg2 = sc_aggregate(hs6.reshape((R + 1) * N, F), gidx, dst, coef)
    s, c, a = _bn_consts(p)
    return conv_epilogue(agg2, hs6[R], s, c, a, h if residual else None)


def kernel(x, edge_index, edge_attr, params):
    src = edge_index[0].astype(jnp.int32)
    dst = edge_index[1].astype(jnp.int32)
    et = edge_attr[:, 4].astype(jnp.int32)

    pad = NSBP * SB - E
    hidx = jnp.concatenate(
        [dst * R + et, jnp.full((pad,), HB * SB - 1, jnp.int32)])
    gidx = jnp.concatenate([et * N + src, jnp.zeros((pad,), jnp.int32)])
    dst = jnp.concatenate([dst, jnp.zeros((pad,), jnp.int32)])

    p0, p1 = sc_histogram(hidx)
    recip = tc_merge_recip(p0.reshape(HB, SB),
                           p1.reshape(HB, SB)).reshape(HB * SB)
    coef = sc_coef(recip, hidx)

    h = _conv_block(x, gidx, dst, coef, params["enc0"], residual=False)
    h = _conv_block(h, gidx, dst, coef, params["enc1"], residual=True)

    wm = jnp.concatenate([params["fc_mu"]["w"], params["fc_logvar"]["w"]], axis=1)
    bm = jnp.concatenate([params["fc_mu"]["b"], params["fc_logvar"]["b"]])
    ml = mm_bias(h, wm, bm.reshape(1, -1))
    mu, logvar = ml[:, :64], ml[:, 64:]

    d = jnp.concatenate([mu, x], axis=1)
    d = _conv_block(d, gidx, dst, coef, params["dec0"], residual=False)
    d = _conv_block(d, gidx, dst, coef, params["dec1"], residual=True)
    out = mm_bias(d, params["fc_out"]["w"], params["fc_out"]["b"].reshape(1, -1))
    return (out, mu, logvar)


# speedup vs baseline: 1.7674x; 1.1136x over previous
import functools

import jax
import jax.numpy as jnp
from jax import lax
from jax.experimental import pallas as pl
from jax.experimental.pallas import tpu as pltpu
from jax.experimental.pallas import tpu_sc as plsc

N = 10000
E = 320000
F = 128
R = 5

ROW_BLK = 1000

NC = 2
NS = 16
L = 16
NW = NC * NS
SB = 128
NSB = E // SB
RPT = 624


def _rel_mm_body(x_ref, w_ref, o_ref):
    o_ref[0] = jnp.dot(x_ref[...], w_ref[0],
                       preferred_element_type=jnp.float32)


def rel_matmul(x, w_all):
    J, Din, Fo = w_all.shape
    n = x.shape[0]
    grid = (J, n // ROW_BLK)
    return pl.pallas_call(
        _rel_mm_body,
        grid=grid,
        in_specs=[
            pl.BlockSpec((ROW_BLK, Din), lambda j, i: (i, 0)),
            pl.BlockSpec((1, Din, Fo), lambda j, i: (j, 0, 0)),
        ],
        out_specs=pl.BlockSpec((1, ROW_BLK, Fo), lambda j, i: (j, i, 0)),
        out_shape=jax.ShapeDtypeStruct((J, n, Fo), jnp.float32),
    )(x, w_all)


def _epilogue_body(agg_ref, root_ref, s_ref, c_ref, a_ref, res_ref, o_ref):
    agg = agg_ref[0] + agg_ref[1]
    t = (agg + root_ref[...]) * s_ref[...] + c_ref[...]
    t = jnp.where(t >= 0, t, a_ref[0, 0] * t)
    o_ref[...] = t + res_ref[...]


def _epilogue_body_nores(agg_ref, root_ref, s_ref, c_ref, a_ref, o_ref):
    agg = agg_ref[0] + agg_ref[1]
    t = (agg + root_ref[...]) * s_ref[...] + c_ref[...]
    o_ref[...] = jnp.where(t >= 0, t, a_ref[0, 0] * t)


def conv_epilogue(agg2, root, s, c, a, res=None):
    n = root.shape[0]
    grid = (n // ROW_BLK,)
    agg_spec = pl.BlockSpec((NC, ROW_BLK, F), lambda i: (0, i, 0))
    row_spec = pl.BlockSpec((ROW_BLK, F), lambda i: (i, 0))
    vec_spec = pl.BlockSpec((1, F), lambda i: (0, 0))
    scal_spec = pl.BlockSpec((1, 1), lambda i: (0, 0))
    if res is None:
        return pl.pallas_call(
            _epilogue_body_nores,
            grid=grid,
            in_specs=[agg_spec, row_spec, vec_spec, vec_spec, scal_spec],
            out_specs=row_spec,
            out_shape=jax.ShapeDtypeStruct((n, F), jnp.float32),
        )(agg2, root, s, c, a)
    return pl.pallas_call(
        _epilogue_body,
        grid=grid,
        in_specs=[agg_spec, row_spec, vec_spec, vec_spec, scal_spec, row_spec],
        out_specs=row_spec,
        out_shape=jax.ShapeDtypeStruct((n, F), jnp.float32),
    )(agg2, root, s, c, a, res)


def _mm_bias_body(x_ref, w_ref, b_ref, o_ref):
    o_ref[...] = jnp.dot(x_ref[...], w_ref[...],
                         preferred_element_type=jnp.float32) + b_ref[...]


def mm_bias(x, w, b):
    n, Din = x.shape
    Fo = w.shape[1]
    return pl.pallas_call(
        _mm_bias_body,
        grid=(n // ROW_BLK,),
        in_specs=[
            pl.BlockSpec((ROW_BLK, Din), lambda i: (i, 0)),
            pl.BlockSpec((Din, Fo), lambda i: (0, 0)),
            pl.BlockSpec((1, Fo), lambda i: (0, 0)),
        ],
        out_specs=pl.BlockSpec((ROW_BLK, Fo), lambda i: (i, 0)),
        out_shape=jax.ShapeDtypeStruct((n, Fo), jnp.float32),
    )(x, w, b)


SBT = 80
NSBP = NW * SBT
NB2 = SBT // 2


def _scale_block(rows_v, coef_all, cb):
    def scale_body(g, c2):
        rb = g * L
        cvec = coef_all[pl.ds(cb + rb, L)]
        for r in range(L):
            bvec = cvec.at[jnp.full((L,), r, jnp.int32)].get(
                mode="promise_in_bounds")
            for q in range(F // L):
                rows_v[rb + r, pl.ds(q * L, L)] = (
                    rows_v[rb + r, pl.ds(q * L, L)] * bvec)
        return c2

    lax.fori_loop(0, SB // L, scale_body, 0)


HB = 512
EPT = NSBP * SB // NW
HPS = HB * SB // NS
HCB = EPT // SB


def _sc_hist_body(hidx, out0, out1, hbuf0, hbuf1, ones_v, zeros_v, acc_sh,
                  ssem0, ssem1):
    cid = lax.axis_index("c")
    sid = lax.axis_index("s")
    w = sid * NC + cid

    one = jnp.ones((L,), jnp.float32)
    zero = jnp.zeros((L,), jnp.float32)

    def fill_body(g, carry):
        ones_v[pl.ds(g * L, L)] = one
        zeros_v[pl.ds(g * L, L)] = zero
        return carry

    lax.fori_loop(0, SB // L, fill_body, 0)

    base = sid * HPS
    for k in range(HPS // SB):
        pltpu.sync_copy(zeros_v, acc_sh.at[pl.ds(base + k * SB, SB)])
    plsc.subcore_barrier()

    ebs = w * EPT

    def load_idx(j, hb):
        pltpu.sync_copy(hidx.at[pl.ds(ebs + j * SB, SB)], hb)

    def scat(hb, sem):
        return pltpu.async_copy(ones_v, acc_sh.at[hb], sem, add=True)

    load_idx(0, hbuf0)
    scat(hbuf0, ssem0)
    load_idx(1, hbuf1)
    scat(hbuf1, ssem1)

    def body2(i, carry):
        a = 2 * i
        b = a + 1
        pltpu.make_async_copy(ones_v, acc_sh.at[hbuf0], ssem0).wait()

        @pl.when(a + 2 < HCB)
        def _s0():
            load_idx(a + 2, hbuf0)
            scat(hbuf0, ssem0)

        pltpu.make_async_copy(ones_v, acc_sh.at[hbuf1], ssem1).wait()

        @pl.when(b + 2 < HCB)
        def _s1():
            load_idx(b + 2, hbuf1)
            scat(hbuf1, ssem1)

        return carry

    lax.fori_loop(0, HCB // 2, body2, 0)
    plsc.subcore_barrier()

    @pl.when(cid == 0)
    def _co0():
        pltpu.sync_copy(acc_sh.at[pl.ds(base, HPS)],
                        out0.at[pl.ds(base, HPS)])

    @pl.when(cid == 1)
    def _co1():
        pltpu.sync_copy(acc_sh.at[pl.ds(base, HPS)],
                        out1.at[pl.ds(base, HPS)])


def sc_histogram(hidx):
    mesh = plsc.VectorSubcoreMesh(core_axis_name="c", subcore_axis_name="s",
                                  num_cores=NC, num_subcores=NS)
    return pl.kernel(
        _sc_hist_body,
        out_type=(jax.ShapeDtypeStruct((HB * SB,), jnp.float32),
                  jax.ShapeDtypeStruct((HB * SB,), jnp.float32)),
        mesh=mesh,
        scratch_types=[
            pltpu.VMEM((SB,), jnp.int32),
            pltpu.VMEM((SB,), jnp.int32),
            pltpu.VMEM((SB,), jnp.float32),
            pltpu.VMEM((SB,), jnp.float32),
            pltpu.VMEM_SHARED((HB * SB,), jnp.float32),
            pltpu.SemaphoreType.DMA,
            pltpu.SemaphoreType.DMA,
        ],
    )(hidx)


def _merge_body(p0_ref, p1_ref, o_ref):
    s = p0_ref[...] + p1_ref[...]
    row = lax.broadcasted_iota(jnp.int32, (HB, SB), 0)
    col = lax.broadcasted_iota(jnp.int32, (HB, SB), 1)
    b = row * SB + col
    o_ref[...] = jnp.where(b < N * R, 1.0 / jnp.maximum(s, 1.0), 0.0)


def tc_merge_recip(p0, p1):
    return pl.pallas_call(
        _merge_body,
        out_shape=jax.ShapeDtypeStruct((HB, SB), jnp.float32),
    )(p0, p1)


def _sc_coef_body(recip, hidx, out, hbuf0, hbuf1, cbuf0, cbuf1,
                  gsem0, gsem1):
    cid = lax.axis_index("c")
    sid = lax.axis_index("s")
    w = sid * NC + cid
    ebs = w * EPT

    def load_idx(j, hb):
        pltpu.sync_copy(hidx.at[pl.ds(ebs + j * SB, SB)], hb)

    def gather(hb, cb, sem):
        return pltpu.async_copy(recip.at[hb], cb, sem)

    def flush(j, cb):
        pltpu.sync_copy(cb, out.at[pl.ds(ebs + j * SB, SB)])

    load_idx(0, hbuf0)
    gather(hbuf0, cbuf0, gsem0)
    load_idx(1, hbuf1)
    gather(hbuf1, cbuf1, gsem1)

    def body2(i, carry):
        a = 2 * i
        b = a + 1
        pltpu.make_async_copy(recip.at[hbuf0], cbuf0, gsem0).wait()
        flush(a, cbuf0)

        @pl.when(a + 2 < SBT)
        def _g0():
            load_idx(a + 2, hbuf0)
            gather(hbuf0, cbuf0, gsem0)

        pltpu.make_async_copy(recip.at[hbuf1], cbuf1, gsem1).wait()
        flush(b, cbuf1)

        @pl.when(b + 2 < SBT)
        def _g1():
            load_idx(b + 2, hbuf1)
            gather(hbuf1, cbuf1, gsem1)

        return carry

    lax.fori_loop(0, NB2, body2, 0)


def sc_coef(recip_flat, hidx):
    mesh = plsc.VectorSubcoreMesh(core_axis_name="c", subcore_axis_name="s",
                                  num_cores=NC, num_subcores=NS)
    return pl.kernel(
        _sc_coef_body,
        out_type=jax.ShapeDtypeStruct((NSBP * SB,), jnp.float32),
        mesh=mesh,
        scratch_types=[
            pltpu.VMEM((SB,), jnp.int32),
            pltpu.VMEM((SB,), jnp.int32),
            pltpu.VMEM((SB,), jnp.float32),
            pltpu.VMEM((SB,), jnp.float32),
            pltpu.SemaphoreType.DMA,
            pltpu.SemaphoreType.DMA,
        ],
    )(recip_flat, hidx)


def _sc_agg_body(hs, gidx, dst, coef, out,
                 gidx0, dst0, coef0, gidx1, dst1, coef1,
                 gidx2s, dst2s, coef2s,
                 rows0, rows1, rows2, agg_sh,
                 gsem0, gsem1, gsem2,
                 ssem0, ssem1, ssem2):
    cid = lax.axis_index("c")
    sid = lax.axis_index("s")
    w = sid * NC + cid

    zero = jnp.zeros((L,), jnp.float32)

    def zero_body(r, carry):
        for g in range(F // L):
            rows0[r, pl.ds(g * L, L)] = zero
        return carry

    lax.fori_loop(0, SB, zero_body, 0)

    base = sid * RPT
    for k in range(4):
        pltpu.sync_copy(rows0.at[pl.ds(0, SB)],
                        agg_sh.at[pl.ds(base + k * SB, SB)])
    pltpu.sync_copy(rows0.at[pl.ds(0, 112)],
                    agg_sh.at[pl.ds(base + 4 * SB, 112)])

    @pl.when(sid == NS - 1)
    def _zero_tail():
        pltpu.sync_copy(rows0.at[pl.ds(0, 16)],
                        agg_sh.at[pl.ds(NS * RPT, 16)])

    plsc.subcore_barrier()

    ebs = w * SBT * SB

    def load_idx(j, gv, dv, cv):
        eb = ebs + j * SB
        pltpu.sync_copy(gidx.at[pl.ds(eb, SB)], gv)
        pltpu.sync_copy(dst.at[pl.ds(eb, SB)], dv)
        pltpu.sync_copy(coef.at[pl.ds(eb, SB)], cv)

    def gather(gv, rows, sem):
        return pltpu.async_copy(hs.at[gv], rows, sem)

    def wait_gather(gv, rows, sem):
        pltpu.make_async_copy(hs.at[gv], rows, sem).wait()

    def scatter(dv, rows, sem):
        return pltpu.async_copy(rows, agg_sh.at[dv], sem, add=True)

    def wait_scatter(dv, rows, sem):
        pltpu.make_async_copy(rows, agg_sh.at[dv], sem).wait()

    load_idx(0, gidx0, dst0, coef0)
    gather(gidx0, rows0, gsem0)
    load_idx(1, gidx1, dst1, coef1)
    gather(gidx1, rows1, gsem1)
    load_idx(2, gidx2s, dst2s, coef2s)
    gather(gidx2s, rows2, gsem2)

    NB3 = (SBT - 2) // 3

    def body3(i, carry):
        a = 3 * i
        wait_gather(gidx0, rows0, gsem0)
        _scale_block(rows0, coef0, 0)
        scatter(dst0, rows0, ssem0)

        wait_gather(gidx1, rows1, gsem1)
        _scale_block(rows1, coef1, 0)
        scatter(dst1, rows1, ssem1)
        wait_scatter(dst0, rows0, ssem0)

        @pl.when(a + 3 < SBT)
        def _g0():
            load_idx(a + 3, gidx0, dst0, coef0)
            gather(gidx0, rows0, gsem0)

        wait_gather(gidx2s, rows2, gsem2)
        _scale_block(rows2, coef2s, 0)
        scatter(dst2s, rows2, ssem2)
        wait_scatter(dst1, rows1, ssem1)

        @pl.when(a + 4 < SBT)
        def _g1():
            load_idx(a + 4, gidx1, dst1, coef1)
            gather(gidx1, rows1, gsem1)

        wait_scatter(dst2s, rows2, ssem2)

        @pl.when(a + 5 < SBT)
        def _g2():
            load_idx(a + 5, gidx2s, dst2s, coef2s)
            gather(gidx2s, rows2, gsem2)

        return carry

    lax.fori_loop(0, NB3, body3, 0)

    wait_gather(gidx0, rows0, gsem0)
    _scale_block(rows0, coef0, 0)
    scatter(dst0, rows0, ssem0)
    wait_gather(gidx1, rows1, gsem1)
    _scale_block(rows1, coef1, 0)
    scatter(dst1, rows1, ssem1)
    wait_scatter(dst0, rows0, ssem0)
    wait_scatter(dst1, rows1, ssem1)
    plsc.subcore_barrier()

    for k in range(4):
        pltpu.sync_copy(agg_sh.at[pl.ds(base + k * SB, SB)],
                        out.at[cid, pl.ds(base + k * SB, SB), :])
    pltpu.sync_copy(agg_sh.at[pl.ds(base + 4 * SB, 112)],
                    out.at[cid, pl.ds(base + 4 * SB, 112), :])

    @pl.when(sid == NS - 1)
    def _out_tail():
        pltpu.sync_copy(agg_sh.at[pl.ds(NS * RPT, 16)],
                        out.at[cid, pl.ds(NS * RPT, 16), :])


def sc_aggregate(hs_flat, gidx2, dst2, coef2):
    mesh = plsc.VectorSubcoreMesh(core_axis_name="c", subcore_axis_name="s",
                                  num_cores=NC, num_subcores=NS)
    return pl.kernel(
        _sc_agg_body,
        out_type=jax.ShapeDtypeStruct((NC, N, F), jnp.float32),
        mesh=mesh,
        scratch_types=[
            pltpu.VMEM((SB,), jnp.int32),
            pltpu.VMEM((SB,), jnp.int32),
            pltpu.VMEM((SB,), jnp.float32),
            pltpu.VMEM((SB,), jnp.int32),
            pltpu.VMEM((SB,), jnp.int32),
            pltpu.VMEM((SB,), jnp.float32),
            pltpu.VMEM((SB,), jnp.int32),
            pltpu.VMEM((SB,), jnp.int32),
            pltpu.VMEM((SB,), jnp.float32),
            pltpu.VMEM((SB, F), jnp.float32),
            pltpu.VMEM((SB, F), jnp.float32),
            pltpu.VMEM((SB, F), jnp.float32),
            pltpu.VMEM_SHARED((N, F), jnp.float32),
            pltpu.SemaphoreType.DMA,
            pltpu.SemaphoreType.DMA,
            pltpu.SemaphoreType.DMA,
            pltpu.SemaphoreType.DMA,
            pltpu.SemaphoreType.DMA,
            pltpu.SemaphoreType.DMA,
        ],
    )(hs_flat, gidx2, dst2, coef2)


def _bn_consts(p):
    s = p["bn_g"] / jnp.sqrt(p["bn_rv"] + 1e-5)
    c = p["bn_b"] + (p["b"] - p["bn_rm"]) * s
    return s.reshape(1, F), c.reshape(1, F), p["prelu"].reshape(1, 1)


def _conv_block(h, gidx, dst, coef, p, residual):
    w_all = jnp.concatenate([p["w_rel"], p["w_root"][None]], axis=0)
    hs6 = rel_matmul(h, w_all)
    agg2 = sc_aggregate(hs6.reshape((R + 1) * N, F), gidx, dst, coef)
    s, c, a = _bn_consts(p)
    return conv_epilogue(agg2, hs6[R], s, c, a, h if residual else None)


def kernel(x, edge_index, edge_attr, params):
    src = edge_index[0].astype(jnp.int32)
    dst = edge_index[1].astype(jnp.int32)
    et = edge_attr[:, 4].astype(jnp.int32)

    pad = NSBP * SB - E
    hidx = jnp.concatenate(
        [dst * R + et, jnp.full((pad,), HB * SB - 1, jnp.int32)])
    gidx = jnp.concatenate([et * N + src, jnp.zeros((pad,), jnp.int32)])
    dst = jnp.concatenate([dst, jnp.zeros((pad,), jnp.int32)])

    p0, p1 = sc_histogram(hidx)
    recip = tc_merge_recip(p0.reshape(HB, SB),
                           p1.reshape(HB, SB)).reshape(HB * SB)
    coef = sc_coef(recip, hidx)

    h = _conv_block(x, gidx, dst, coef, params["enc0"], residual=False)
    h = _conv_block(h, gidx, dst, coef, params["enc1"], residual=True)

    wm = jnp.concatenate([params["fc_mu"]["w"], params["fc_logvar"]["w"]], axis=1)
    bm = jnp.concatenate([params["fc_mu"]["b"], params["fc_logvar"]["b"]])
    ml = mm_bias(h, wm, bm.reshape(1, -1))
    mu, logvar = ml[:, :64], ml[:, 64:]

    d = jnp.concatenate([mu, x], axis=1)
    d = _conv_block(d, gidx, dst, coef, params["dec0"], residual=False)
    d = _conv_block(d, gidx, dst, coef, params["dec1"], residual=True)
    out = mm_bias(d, params["fc_out"]["w"], params["fc_out"]["b"].reshape(1, -1))
    return (out, mu, logvar)
